# Initial kernel scaffold; baseline (speedup 1.0000x reference)
import jax
import jax.numpy as jnp
from jax.experimental import pallas as pl


def kernel(z_e, codebook):
    raise NotImplementedError("write your pallas kernel here")


# speedup vs baseline: 27.6242x; 27.6242x over previous
import jax
import jax.numpy as jnp
from jax.experimental import pallas as pl

_N = 4 * 24 * 24
_D = 256
_K = 512


def _vq_body(z_ref, ct_ref, out_ref):
    z = z_ref[...]
    ct = ct_ref[...]
    scores = jax.lax.dot_general(
        z, ct, (((1,), (0,)), ((), ())),
        precision=jax.lax.Precision.HIGHEST,
        preferred_element_type=jnp.float32)
    cnorm = jnp.sum(ct * ct, axis=0, keepdims=True)
    d = cnorm - 2.0 * scores
    m = jnp.min(d, axis=1, keepdims=True)
    iota = jax.lax.broadcasted_iota(jnp.int32, d.shape, 1)
    idx = jnp.min(jnp.where(d == m, iota, _K), axis=1, keepdims=True)
    out_ref[...] = idx


def kernel(z_e, codebook):
    z2d = z_e.reshape(_N, _D)
    ct = codebook.T
    out = pl.pallas_call(
        _vq_body,
        out_shape=jax.ShapeDtypeStruct((_N, 1), jnp.int32),
    )(z2d, ct)
    return out.reshape(z_e.shape[:3])


# speedup vs baseline: 41.9766x; 1.5196x over previous
import jax
import jax.numpy as jnp
from jax.experimental import pallas as pl
from jax.experimental.pallas import tpu as pltpu

_B = 4
_HW = 24 * 24
_D = 256
_K = 512


def _vq_body(z_ref, c_ref, out_ref, ct_ref, bias_ref):
    @pl.when(pl.program_id(0) == 0)
    def _():
        ct = jnp.transpose(c_ref[...])
        ct_ref[...] = ct
        bias_ref[...] = 0.5 * jnp.sum(ct * ct, axis=0, keepdims=True)

    scores = jax.lax.dot_general(
        z_ref[...], ct_ref[...], (((1,), (0,)), ((), ())),
        precision=jax.lax.Precision.HIGHEST,
        preferred_element_type=jnp.float32)
    d = bias_ref[...] - scores
    m = jnp.min(d, axis=1, keepdims=True)
    iota = jax.lax.broadcasted_iota(jnp.int32, d.shape, 1)
    idx = jnp.min(jnp.where(d == m, iota, _K), axis=1)
    out_ref[...] = idx.reshape(1, 24, 24)


def kernel(z_e, codebook):
    z2d = z_e.reshape(_B * _HW, _D)
    out = pl.pallas_call(
        _vq_body,
        grid=(_B,),
        in_specs=[
            pl.BlockSpec((_HW, _D), lambda b: (b, 0)),
            pl.BlockSpec((_K, _D), lambda b: (0, 0)),
        ],
        out_specs=pl.BlockSpec((1, 24, 24), lambda b: (b, 0, 0)),
        out_shape=jax.ShapeDtypeStruct((_B, 24, 24), jnp.int32),
        scratch_shapes=[
            pltpu.VMEM((_D, _K), jnp.float32),
            pltpu.VMEM((1, _K), jnp.float32),
        ],
    )(z2d, codebook)
    return out
